# baseline (speedup 1.0000x reference)
import functools

import jax
import jax.numpy as jnp
from jax import lax
from jax.experimental import pallas as pl
from jax.experimental.pallas import tpu as pltpu
from jax.experimental.pallas import tpu_sc as plsc

N_NODES = 10000
D_FEAT = 128
N_EDGES = 320000

NC = 2
NS = 16
NW = NC * NS
CHUNK = 120
NITER = 84
EPW = NITER * CHUNK
E_PAD = NW * EPW
N_PAD = 10240
RPN = N_PAD // NS
NRB = 3
NIB = 6
NQ = NITER // NIB


def _sc_segment_sum(x, idx4, zeros):
    mesh = plsc.VectorSubcoreMesh(core_axis_name="c", subcore_axis_name="s")

    @functools.partial(
        pl.kernel,
        out_type=jax.ShapeDtypeStruct((NC, N_PAD, D_FEAT), jnp.float32),
        mesh=mesh,
        scratch_types=[
            pltpu.VMEM((NIB, 2, CHUNK), jnp.int32),
            pltpu.VMEM((NRB, CHUNK, D_FEAT), jnp.float32),
            pltpu.VMEM_SHARED((N_PAD, D_FEAT), jnp.float32),
            pltpu.SemaphoreType.DMA((NIB,)),
            pltpu.SemaphoreType.DMA((NRB,)),
            pltpu.SemaphoreType.DMA((NRB,)),
        ],
    )
    def k(x_hbm, idx_hbm, z_hbm, out_hbm, idx_v, rows_v, acc,
          isem, gsem, ssem):
        c = lax.axis_index("c")
        s = lax.axis_index("s")
        w = c * NS + s

        pltpu.sync_copy(z_hbm.at[pl.ds(s * RPN, RPN)],
                        acc.at[pl.ds(s * RPN, RPN)])

        def start_fetch(i, r):
            pltpu.async_copy(idx_hbm.at[w].at[i], idx_v.at[r], isem.at[r])

        def wait_fetch(r):
            pltpu.make_async_copy(idx_hbm.at[w].at[0], idx_v.at[r],
                                  isem.at[r]).wait()

        def start_gather(r, b):
            pltpu.async_copy(x_hbm.at[idx_v.at[r].at[0]], rows_v.at[b],
                             gsem.at[b])

        def wait_gather(b):
            pltpu.make_async_copy(x_hbm.at[idx_v.at[0].at[0]],
                                  rows_v.at[b], gsem.at[b]).wait()

        def start_scatter(r, b):
            pltpu.async_copy(rows_v.at[b], acc.at[idx_v.at[r].at[1]],
                             ssem.at[b], add=True)

        def wait_scatter(b):
            pltpu.make_async_copy(rows_v.at[b], acc.at[idx_v.at[0].at[1]],
                                  ssem.at[b]).wait()

        def slot(i, j, do_fetch, do_gather, do_wait_scatter=True):
            b = j % NRB
            wait_gather(b)
            start_scatter(j % NIB, b)
            if do_gather:
                b1 = (j + 1) % NRB
                if do_wait_scatter:
                    wait_scatter(b1)
                wait_fetch((j + 1) % NIB)
                start_gather((j + 1) % NIB, b1)
            if do_fetch:
                start_fetch(i + 3, (j + 3) % NIB)

        start_fetch(0, 0)
        start_fetch(1, 1)
        start_fetch(2, 2)
        wait_fetch(0)
        start_gather(0, 0)
        plsc.subcore_barrier()

        for j in range(NIB):
            slot(j, j, True, True, do_wait_scatter=(j >= 2))

        @pl.loop(1, NQ - 1)
        def _(q):
            i0 = q * NIB
            for j in range(NIB):
                slot(i0 + j, j, True, True)

        i0 = (NQ - 1) * NIB
        for j in range(NIB):
            slot(i0 + j, j, do_fetch=(j < 3), do_gather=(j < 5))
        for b in range(NRB):
            wait_scatter(b)

        plsc.subcore_barrier()
        pltpu.sync_copy(acc.at[pl.ds(s * RPN, RPN)],
                        out_hbm.at[c].at[pl.ds(s * RPN, RPN)])

    return k(x, idx4, zeros)


def _tc_dense(x, p0, p1, W_relT, b_rel, W_rootT):
    def body(x_ref, p0_ref, p1_ref, wr_ref, b_ref, wt_ref, o_ref):
        aggr = p0_ref[...] + p1_ref[...]
        v = (jnp.dot(aggr, wr_ref[...], preferred_element_type=jnp.float32)
             + jnp.dot(x_ref[...], wt_ref[...],
                       preferred_element_type=jnp.float32)
             + b_ref[...])
        o_ref[...] = jnp.where(v > 0, v, 0.01 * v)

    return pl.pallas_call(
        body,
        out_shape=jax.ShapeDtypeStruct((N_NODES, D_FEAT), jnp.float32),
    )(x, p0, p1, W_relT, b_rel, W_rootT)


def kernel(x, edge_index, W_rel0, b_rel0, W_root0, W_rel1, b_rel1, W_root1):
    src = edge_index[0].astype(jnp.int32)
    dst = edge_index[1].astype(jnp.int32)
    pad = E_PAD - N_EDGES
    pad_idx = jnp.arange(pad, dtype=jnp.int32)
    src3 = jnp.concatenate([src, pad_idx % N_NODES]).reshape(NW, NITER, CHUNK)
    dst3 = jnp.concatenate(
        [dst, N_NODES + pad_idx % (N_PAD - N_NODES)]).reshape(NW, NITER, CHUNK)
    idx4 = jnp.stack([src3, dst3], axis=2)
    zeros = jnp.zeros((N_PAD, D_FEAT), jnp.float32)

    p = _sc_segment_sum(x, idx4, zeros)
    h = _tc_dense(x, p[0, :N_NODES], p[1, :N_NODES], W_rel0.T,
                  b_rel0.reshape(1, D_FEAT), W_root0.T)
    p = _sc_segment_sum(h, idx4, zeros)
    return _tc_dense(h, p[0, :N_NODES], p[1, :N_NODES], W_rel1.T,
                     b_rel1.reshape(1, D_FEAT), W_root1.T)


# speedup vs baseline: 1.0669x; 1.0669x over previous
import functools

import jax
import jax.numpy as jnp
from jax import lax
from jax.experimental import pallas as pl
from jax.experimental.pallas import tpu as pltpu
from jax.experimental.pallas import tpu_sc as plsc

N_NODES = 10000
D_FEAT = 128
N_EDGES = 320000

NC = 2
NS = 16
NW = NC * NS
CHUNK = 88
NITER = 116
EPW = NITER * CHUNK
E_PAD = NW * EPW
N_PAD = 10240
RPN = N_PAD // NS
NRB = 4
NIB = 8
NPAIR = NITER // 2


def _sc_segment_sum(x, idx4, zeros):
    mesh = plsc.VectorSubcoreMesh(core_axis_name="c", subcore_axis_name="s")

    @functools.partial(
        pl.kernel,
        out_type=jax.ShapeDtypeStruct((NC, N_PAD, D_FEAT), jnp.float32),
        mesh=mesh,
        scratch_types=[
            pltpu.VMEM((NIB, 2, CHUNK), jnp.int32),
            pltpu.VMEM((NRB, CHUNK, D_FEAT), jnp.float32),
            pltpu.VMEM_SHARED((N_PAD, D_FEAT), jnp.float32),
            pltpu.SemaphoreType.DMA((NIB,)),
            pltpu.SemaphoreType.DMA((NRB,)),
            pltpu.SemaphoreType.DMA((NRB,)),
        ],
    )
    def k(x_hbm, idx_hbm, z_hbm, out_hbm, idx_v, rows_v, acc,
          isem, gsem, ssem):
        c = lax.axis_index("c")
        s = lax.axis_index("s")
        w = c * NS + s

        pltpu.sync_copy(z_hbm.at[pl.ds(s * RPN, RPN)],
                        acc.at[pl.ds(s * RPN, RPN)])

        def start_fetch(i, r):
            pltpu.async_copy(idx_hbm.at[w].at[i], idx_v.at[r], isem.at[r])

        def wait_fetch(r):
            pltpu.make_async_copy(idx_hbm.at[w].at[0], idx_v.at[r],
                                  isem.at[r]).wait()

        def start_gather(r, b):
            pltpu.async_copy(x_hbm.at[idx_v.at[r].at[0]], rows_v.at[b],
                             gsem.at[b])

        def wait_gather(b):
            pltpu.make_async_copy(x_hbm.at[idx_v.at[0].at[0]],
                                  rows_v.at[b], gsem.at[b]).wait()

        def start_scatter(r, b):
            pltpu.async_copy(rows_v.at[b], acc.at[idx_v.at[r].at[1]],
                             ssem.at[b], add=True)

        def wait_scatter(b):
            pltpu.make_async_copy(rows_v.at[b], acc.at[idx_v.at[0].at[1]],
                                  ssem.at[b]).wait()

        def pair(p, P, do_fetch, do_gather, do_wait=True):
            r0 = (2 * P) % NIB
            b0 = (2 * P) % NRB
            wait_gather(b0)
            wait_gather(b0 + 1)
            start_scatter(r0, b0)
            start_scatter(r0 + 1, b0 + 1)
            if do_gather:
                if do_wait:
                    wait_scatter((b0 + 2) % NRB)
                    wait_scatter((b0 + 3) % NRB)
                wait_fetch((r0 + 2) % NIB)
                start_gather((r0 + 2) % NIB, (b0 + 2) % NRB)
                wait_fetch((r0 + 3) % NIB)
                start_gather((r0 + 3) % NIB, (b0 + 3) % NRB)
            if do_fetch:
                start_fetch(2 * p + 4, (r0 + 4) % NIB)
                start_fetch(2 * p + 5, (r0 + 5) % NIB)

        for r in range(4):
            start_fetch(r, r)
        wait_fetch(0)
        start_gather(0, 0)
        wait_fetch(1)
        start_gather(1, 1)
        plsc.subcore_barrier()

        pair(0, 0, True, True, do_wait=False)
        pair(1, 1, True, True)

        @pl.loop(0, 13)
        def _(g):
            p0 = 4 * g + 2
            pair(p0, 2, True, True)
            pair(p0 + 1, 3, True, True)
            pair(p0 + 2, 0, True, True)
            pair(p0 + 3, 1, True, True)

        pair(NPAIR - 4, 2, True, True)
        pair(NPAIR - 3, 3, True, True)
        pair(NPAIR - 2, 0, False, True)
        pair(NPAIR - 1, 1, False, False)
        for b in range(NRB):
            wait_scatter(b)

        plsc.subcore_barrier()
        pltpu.sync_copy(acc.at[pl.ds(s * RPN, RPN)],
                        out_hbm.at[c].at[pl.ds(s * RPN, RPN)])

    return k(x, idx4, zeros)


def _tc_dense(x, p0, p1, W_relT, b_rel, W_rootT):
    def body(x_ref, p0_ref, p1_ref, wr_ref, b_ref, wt_ref, o_ref):
        aggr = p0_ref[...] + p1_ref[...]
        v = (jnp.dot(aggr, wr_ref[...], preferred_element_type=jnp.float32)
             + jnp.dot(x_ref[...], wt_ref[...],
                       preferred_element_type=jnp.float32)
             + b_ref[...])
        o_ref[...] = jnp.where(v > 0, v, 0.01 * v)

    return pl.pallas_call(
        body,
        out_shape=jax.ShapeDtypeStruct((N_NODES, D_FEAT), jnp.float32),
    )(x, p0, p1, W_relT, b_rel, W_rootT)


def kernel(x, edge_index, W_rel0, b_rel0, W_root0, W_rel1, b_rel1, W_root1):
    src = edge_index[0].astype(jnp.int32)
    dst = edge_index[1].astype(jnp.int32)
    pad = E_PAD - N_EDGES
    pad_idx = jnp.arange(pad, dtype=jnp.int32)
    src3 = jnp.concatenate([src, pad_idx % N_NODES]).reshape(NW, NITER, CHUNK)
    dst3 = jnp.concatenate(
        [dst, N_NODES + pad_idx % (N_PAD - N_NODES)]).reshape(NW, NITER, CHUNK)
    idx4 = jnp.stack([src3, dst3], axis=2)
    zeros = jnp.zeros((N_PAD, D_FEAT), jnp.float32)

    p = _sc_segment_sum(x, idx4, zeros)
    h = _tc_dense(x, p[0, :N_NODES], p[1, :N_NODES], W_rel0.T,
                  b_rel0.reshape(1, D_FEAT), W_root0.T)
    p = _sc_segment_sum(h, idx4, zeros)
    return _tc_dense(h, p[0, :N_NODES], p[1, :N_NODES], W_rel1.T,
                     b_rel1.reshape(1, D_FEAT), W_root1.T)


# speedup vs baseline: 1.2668x; 1.1874x over previous
import functools

import jax
import jax.numpy as jnp
from jax import lax
from jax.experimental import pallas as pl
from jax.experimental.pallas import tpu as pltpu
from jax.experimental.pallas import tpu_sc as plsc

N_NODES = 10000
D_FEAT = 128
N_EDGES = 320000

NC = 2
NS = 16
NW = NC * NS
CHUNK = 128
NITER = 80
EPW = NITER * CHUNK
E_PAD = NW * EPW
N_PAD = 10240
RPN = N_PAD // NS
NIB = 4
NQ = NITER // 4


def _sc_segment_sum(x, idx4, zeros):
    mesh = plsc.VectorSubcoreMesh(core_axis_name="c", subcore_axis_name="s")

    @functools.partial(
        pl.kernel,
        out_type=jax.ShapeDtypeStruct((NC, N_PAD, D_FEAT), jnp.float32),
        mesh=mesh,
        scratch_types=[
            pltpu.VMEM((NIB, 2, CHUNK), jnp.int32),
            pltpu.VMEM((2, CHUNK, D_FEAT), jnp.float32),
            pltpu.VMEM_SHARED((N_PAD, D_FEAT), jnp.float32),
            pltpu.SemaphoreType.DMA((NIB,)),
            pltpu.SemaphoreType.DMA((2,)),
            pltpu.SemaphoreType.DMA,
        ],
    )
    def k(x_hbm, idx_hbm, z_hbm, out_hbm, idx_v, rows_v, acc,
          isem, gsem, ssem):
        c = lax.axis_index("c")
        s = lax.axis_index("s")
        w = c * NS + s

        pltpu.sync_copy(z_hbm.at[pl.ds(s * RPN, RPN)],
                        acc.at[pl.ds(s * RPN, RPN)])

        def start_fetch(i, r):
            pltpu.async_copy(idx_hbm.at[w].at[i], idx_v.at[r], isem.at[r])

        def wait_fetch(r):
            pltpu.make_async_copy(idx_hbm.at[w].at[0], idx_v.at[r],
                                  isem.at[r]).wait()

        def start_gather(r, b):
            pltpu.async_copy(x_hbm.at[idx_v.at[r].at[0]], rows_v.at[b],
                             gsem.at[b])

        def wait_gather(b):
            pltpu.make_async_copy(x_hbm.at[idx_v.at[0].at[0]],
                                  rows_v.at[b], gsem.at[b]).wait()

        def scatter(r, b):
            pltpu.async_copy(rows_v.at[b], acc.at[idx_v.at[r].at[1]],
                             ssem, add=True).wait()

        def slot(i, j, do_fetch, do_gather):
            if do_gather:
                wait_fetch((j + 1) % NIB)
                start_gather((j + 1) % NIB, (j + 1) % 2)
            if do_fetch:
                start_fetch(i + 3, (j + 3) % NIB)
            wait_gather(j % 2)
            scatter(j % NIB, j % 2)

        start_fetch(0, 0)
        start_fetch(1, 1)
        start_fetch(2, 2)
        wait_fetch(0)
        start_gather(0, 0)
        plsc.subcore_barrier()

        for j in range(4):
            slot(j, j, True, True)

        @pl.loop(1, NQ - 1)
        def _(q):
            i0 = q * 4
            for j in range(4):
                slot(i0 + j, j, True, True)

        i0 = (NQ - 1) * 4
        slot(i0, 0, True, True)
        slot(i0 + 1, 1, False, True)
        slot(i0 + 2, 2, False, True)
        slot(i0 + 3, 3, False, False)

        plsc.subcore_barrier()
        pltpu.sync_copy(acc.at[pl.ds(s * RPN, RPN)],
                        out_hbm.at[c].at[pl.ds(s * RPN, RPN)])

    return k(x, idx4, zeros)


def _tc_dense(x, p, W_relT, b_rel, W_rootT):
    def body(x_ref, p_ref, wr_ref, b_ref, wt_ref, o_ref):
        aggr = p_ref[0] + p_ref[1]
        rel = jnp.dot(aggr, wr_ref[...], preferred_element_type=jnp.float32)
        v = (rel[:N_NODES]
             + jnp.dot(x_ref[...], wt_ref[...],
                       preferred_element_type=jnp.float32)
             + b_ref[...])
        o_ref[...] = jnp.where(v > 0, v, 0.01 * v)

    return pl.pallas_call(
        body,
        out_shape=jax.ShapeDtypeStruct((N_NODES, D_FEAT), jnp.float32),
    )(x, p, W_relT, b_rel, W_rootT)


def kernel(x, edge_index, W_rel0, b_rel0, W_root0, W_rel1, b_rel1, W_root1):
    src = edge_index[0].astype(jnp.int32)
    dst = edge_index[1].astype(jnp.int32)
    pad = E_PAD - N_EDGES
    pad_idx = jnp.arange(pad, dtype=jnp.int32)
    src3 = jnp.concatenate([src, pad_idx % N_NODES]).reshape(NW, NITER, CHUNK)
    dst3 = jnp.concatenate(
        [dst, N_NODES + pad_idx % (N_PAD - N_NODES)]).reshape(NW, NITER, CHUNK)
    idx4 = jnp.stack([src3, dst3], axis=2)
    zeros = jnp.zeros((N_PAD, D_FEAT), jnp.float32)

    p = _sc_segment_sum(x, idx4, zeros)
    h = _tc_dense(x, p, W_rel0.T, b_rel0.reshape(1, D_FEAT), W_root0.T)
    p = _sc_segment_sum(h, idx4, zeros)
    return _tc_dense(h, p, W_rel1.T, b_rel1.reshape(1, D_FEAT), W_root1.T)
